# Initial kernel scaffold; baseline (speedup 1.0000x reference)
import jax
import jax.numpy as jnp
from jax.experimental import pallas as pl


def kernel(x, embd_weights):
    raise NotImplementedError("write your pallas kernel here")


# speedup vs baseline: 1.6227x; 1.6227x over previous
import functools

import jax
import jax.numpy as jnp
from jax import lax
from jax.experimental import pallas as pl
from jax.experimental.pallas import tpu as pltpu
from jax.experimental.pallas import tpu_sc as plsc

VOCAB = 100000
D_MODEL = 2048
SEQ = 8192

_NUM_CORES = 2
_NUM_SUBCORES = 16
_NW = _NUM_CORES * _NUM_SUBCORES
_PER_W = SEQ // _NW
_CHUNK = 16
_NCHUNK = _PER_W // _CHUNK


def _gather_kernel(table_hbm, idx_hbm, out_hbm, idx_v, buf0, buf1,
                   gsem0, gsem1, osem0, osem1):
    wid = lax.axis_index("s") * _NUM_CORES + lax.axis_index("c")
    base = wid * _PER_W

    pltpu.sync_copy(idx_hbm.at[wid], idx_v)

    bufs = (buf0, buf1)
    gsems = (gsem0, gsem1)
    osems = (osem0, osem1)

    gathers = [None, None]
    outcopies = [None, None]

    gathers[0] = pltpu.async_copy(table_hbm.at[idx_v.at[0]], bufs[0], gsems[0])

    for j in range(_NCHUNK):
        b = j & 1
        nb = (j + 1) & 1
        if j + 1 < _NCHUNK:
            if outcopies[nb] is not None:
                outcopies[nb].wait()
            gathers[nb] = pltpu.async_copy(
                table_hbm.at[idx_v.at[j + 1]], bufs[nb], gsems[nb])
        gathers[b].wait()
        outcopies[b] = pltpu.async_copy(
            bufs[b], out_hbm.at[pl.ds(base + j * _CHUNK, _CHUNK)], osems[b])

    outcopies[0].wait()
    outcopies[1].wait()


@functools.partial(jax.jit, static_argnums=())
def _embedding_gather(idx, table):
    mesh = plsc.VectorSubcoreMesh(core_axis_name="c", subcore_axis_name="s")
    k = functools.partial(
        pl.kernel,
        mesh=mesh,
        out_type=jax.ShapeDtypeStruct((SEQ, D_MODEL), jnp.float32),
        scratch_types=[
            pltpu.VMEM((_NCHUNK, _CHUNK), jnp.int32),
            pltpu.VMEM((_CHUNK, D_MODEL), jnp.float32),
            pltpu.VMEM((_CHUNK, D_MODEL), jnp.float32),
            pltpu.SemaphoreType.DMA,
            pltpu.SemaphoreType.DMA,
            pltpu.SemaphoreType.DMA,
            pltpu.SemaphoreType.DMA,
        ],
    )(_gather_kernel)
    return k(table, idx)


def kernel(x, embd_weights):
    idx = jnp.reshape(x.astype(jnp.int32), (_NW, _NCHUNK, _CHUNK))
    out = _embedding_gather(idx, embd_weights)
    return jnp.reshape(out, (1, 1, SEQ, D_MODEL))


# speedup vs baseline: 1.6580x; 1.0217x over previous
import functools

import jax
import jax.numpy as jnp
from jax import lax
from jax.experimental import pallas as pl
from jax.experimental.pallas import tpu as pltpu
from jax.experimental.pallas import tpu_sc as plsc

VOCAB = 100000
D_MODEL = 2048
SEQ = 8192

_NUM_CORES = 2
_NUM_SUBCORES = 16
_NW = _NUM_CORES * _NUM_SUBCORES
_PER_W = SEQ // _NW
_CHUNK = 16
_NCHUNK = _PER_W // _CHUNK
_NBUF = 3


def _gather_kernel(table_hbm, idx_hbm, out_hbm, idx_v, buf0, buf1, buf2,
                   gsem0, gsem1, gsem2, osem0, osem1, osem2):
    wid = lax.axis_index("s") * _NUM_CORES + lax.axis_index("c")
    base = wid * _PER_W

    pltpu.sync_copy(idx_hbm.at[wid], idx_v)

    bufs = (buf0, buf1, buf2)
    gsems = (gsem0, gsem1, gsem2)
    osems = (osem0, osem1, osem2)

    gathers = [None] * _NBUF
    outcopies = [None] * _NBUF

    for j in range(_NBUF - 1):
        gathers[j] = pltpu.async_copy(
            table_hbm.at[idx_v.at[j]], bufs[j], gsems[j])

    for j in range(_NCHUNK):
        b = j % _NBUF
        nxt = j + _NBUF - 1
        if nxt < _NCHUNK:
            nb = nxt % _NBUF
            if outcopies[nb] is not None:
                outcopies[nb].wait()
            gathers[nb] = pltpu.async_copy(
                table_hbm.at[idx_v.at[nxt]], bufs[nb], gsems[nb])
        gathers[b].wait()
        outcopies[b] = pltpu.async_copy(
            bufs[b], out_hbm.at[pl.ds(base + j * _CHUNK, _CHUNK)], osems[b])

    for b in range(_NBUF):
        if outcopies[b] is not None:
            outcopies[b].wait()


@functools.partial(jax.jit, static_argnums=())
def _embedding_gather(idx, table):
    mesh = plsc.VectorSubcoreMesh(core_axis_name="c", subcore_axis_name="s")
    k = functools.partial(
        pl.kernel,
        mesh=mesh,
        out_type=jax.ShapeDtypeStruct((SEQ, D_MODEL), jnp.float32),
        scratch_types=[
            pltpu.VMEM((_NCHUNK, _CHUNK), jnp.int32),
            pltpu.VMEM((_CHUNK, D_MODEL), jnp.float32),
            pltpu.VMEM((_CHUNK, D_MODEL), jnp.float32),
            pltpu.VMEM((_CHUNK, D_MODEL), jnp.float32),
            pltpu.SemaphoreType.DMA,
            pltpu.SemaphoreType.DMA,
            pltpu.SemaphoreType.DMA,
            pltpu.SemaphoreType.DMA,
            pltpu.SemaphoreType.DMA,
            pltpu.SemaphoreType.DMA,
        ],
    )(_gather_kernel)
    return k(table, idx)


def kernel(x, embd_weights):
    idx = jnp.reshape(x.astype(jnp.int32), (_NW, _NCHUNK, _CHUNK))
    out = _embedding_gather(idx, embd_weights)
    return jnp.reshape(out, (1, 1, SEQ, D_MODEL))
